# baseline (speedup 1.0000x reference)
import functools

import jax
import jax.numpy as jnp
from jax import lax
from jax.experimental import pallas as pl
from jax.experimental.pallas import tpu as pltpu
from jax.experimental.pallas import tpu_sc as plsc

NC = 2
NS = 16
L = 16
NW = NC * NS
B = 16384
BPW = B // NW
CHUNK = 128
NCHUNK = BPW // CHUNK
D = 32
GROUPS = BPW // L
STRIDE = L + 1


def _sc_scores(ri, app_rm, ent_rm):
  mesh = plsc.VectorSubcoreMesh(
      core_axis_name="c", subcore_axis_name="s", num_cores=NC, num_subcores=NS)

  @functools.partial(
      pl.kernel,
      out_type=jax.ShapeDtypeStruct((2, NW, GROUPS, L), jnp.float32),
      mesh=mesh,
      compiler_params=pltpu.CompilerParams(
          needs_layout_passes=False, use_tc_tiling_on_sc=False),
      scratch_types=[
          pltpu.VMEM((NCHUNK, CHUNK), jnp.int32),
          pltpu.VMEM((NCHUNK, CHUNK), jnp.int32),
          pltpu.VMEM((NCHUNK, CHUNK), jnp.int32),
          pltpu.VMEM((NCHUNK, CHUNK), jnp.int32),
          pltpu.VMEM((BPW, D), jnp.bfloat16),
          pltpu.VMEM((BPW, D), jnp.bfloat16),
          pltpu.VMEM((BPW, D), jnp.bfloat16),
          pltpu.VMEM((BPW, D), jnp.bfloat16),
          pltpu.VMEM((BPW * STRIDE,), jnp.float32),
          pltpu.VMEM((BPW * STRIDE,), jnp.float32),
          pltpu.VMEM((GROUPS, L), jnp.float32),
          pltpu.VMEM((GROUPS, L), jnp.float32),
          pltpu.SemaphoreType.DMA,
      ],
  )
  def k(ri_pa, ri_pe, ri_na, ri_ne, app, ent, out_h,
        pa_i, pe_i, na_i, ne_i, pa_r, pe_r, na_r, ne_r,
        sp_flat, sn_flat, s_pos, s_neg, sem):
    wid = lax.axis_index("s") * NC + lax.axis_index("c")

    row0 = wid * NCHUNK
    pltpu.sync_copy(ri_pa.at[pl.ds(row0, NCHUNK)], pa_i)
    pltpu.sync_copy(ri_pe.at[pl.ds(row0, NCHUNK)], pe_i)
    pltpu.sync_copy(ri_na.at[pl.ds(row0, NCHUNK)], na_i)
    pltpu.sync_copy(ri_ne.at[pl.ds(row0, NCHUNK)], ne_i)

    copies = []
    for c in range(NCHUNK):
      dst = pl.ds(c * CHUNK, CHUNK)
      copies.append(pltpu.async_copy(app.at[pa_i.at[c]], pa_r.at[dst], sem))
      copies.append(pltpu.async_copy(ent.at[pe_i.at[c]], pe_r.at[dst], sem))
      copies.append(pltpu.async_copy(app.at[na_i.at[c]], na_r.at[dst], sem))
      copies.append(pltpu.async_copy(ent.at[ne_i.at[c]], ne_r.at[dst], sem))
    for cp in copies:
      cp.wait()

    lane = lax.iota(jnp.int32, L)
    full = pl.ds(0, D)

    def row_halves(ref, r):
      return plsc.unpack(ref[r, full], format=plsc.PackFormat.INTERLEAVED)

    def stage(r, _):
      pa0, pa1 = row_halves(pa_r, r)
      pe0, pe1 = row_halves(pe_r, r)
      na0, na1 = row_halves(na_r, r)
      ne0, ne1 = row_halves(ne_r, r)
      sp_flat[pl.ds(r * STRIDE, L)] = pa0 * pe0 + pa1 * pe1
      sn_flat[pl.ds(r * STRIDE, L)] = na0 * ne0 + na1 * ne1
      return 0

    lax.fori_loop(0, BPW, stage, 0)

    def accum(g, _):
      base = (g * L + lane) * STRIDE
      accp = jnp.zeros((L,), jnp.float32)
      accn = jnp.zeros((L,), jnp.float32)
      for j in range(L):
        accp += plsc.load_gather(sp_flat, [base + j])
        accn += plsc.load_gather(sn_flat, [base + j])
      s_pos[g, :] = accp
      s_neg[g, :] = accn
      return 0

    lax.fori_loop(0, GROUPS, accum, 0)

    pltpu.sync_copy(s_pos, out_h.at[0, wid])
    pltpu.sync_copy(s_neg, out_h.at[1, wid])

  return k(ri[0], ri[1], ri[2], ri[3], app_rm, ent_rm)


def _tc_reduce(scores):
  def body(x_ref, o_ref):
    x = x_ref[...]
    row = lax.broadcasted_iota(jnp.int32, x.shape, 0)
    s = jnp.where(row < 128, x, -x)
    ls = jnp.minimum(s, 0.0) - jnp.log1p(jnp.exp(-jnp.abs(s)))
    o_ref[0, 0] = -jnp.sum(ls)

  out = pl.pallas_call(
      body,
      out_shape=jax.ShapeDtypeStruct((1, 1), jnp.float32),
      out_specs=pl.BlockSpec(memory_space=pltpu.SMEM),
  )(scores)
  return out[0, 0]


def kernel(pos_app, pos_entity, neg_app, neg_entity, app_emb, entity_emb):
  idx = [x.astype(jnp.int32).reshape(B // CHUNK, CHUNK)
         for x in (pos_app, pos_entity, neg_app, neg_entity)]
  scores = _sc_scores(idx, app_emb.astype(jnp.bfloat16),
                      entity_emb.astype(jnp.bfloat16))
  return _tc_reduce(scores.reshape(2 * B // 128, 128))


# speedup vs baseline: 1.1697x; 1.1697x over previous
import functools

import jax
import jax.numpy as jnp
from jax import lax
from jax.experimental import pallas as pl
from jax.experimental.pallas import tpu as pltpu
from jax.experimental.pallas import tpu_sc as plsc

NC = 2
NS = 16
L = 16
NW = NC * NS
B = 16384
BPW = B // NW
CHUNK = 128
NCHUNK = BPW // CHUNK
D = 32
GROUPS = BPW // L
STRIDE = L + 1


def _sc_scores(ri, app_rm, ent_rm):
  mesh = plsc.VectorSubcoreMesh(
      core_axis_name="c", subcore_axis_name="s", num_cores=NC, num_subcores=NS)

  @functools.partial(
      pl.kernel,
      out_type=jax.ShapeDtypeStruct((2, NW, GROUPS, L), jnp.float32),
      mesh=mesh,
      compiler_params=pltpu.CompilerParams(
          needs_layout_passes=False, use_tc_tiling_on_sc=False),
      scratch_types=[
          pltpu.VMEM((NCHUNK, CHUNK), jnp.int32),
          pltpu.VMEM((NCHUNK, CHUNK), jnp.int32),
          pltpu.VMEM((NCHUNK, CHUNK), jnp.int32),
          pltpu.VMEM((NCHUNK, CHUNK), jnp.int32),
          pltpu.VMEM((BPW, D), jnp.float32),
          pltpu.VMEM((BPW, D), jnp.float32),
          pltpu.VMEM((BPW, D), jnp.float32),
          pltpu.VMEM((BPW, D), jnp.float32),
          pltpu.VMEM((BPW * STRIDE,), jnp.float32),
          pltpu.VMEM((BPW * STRIDE,), jnp.float32),
          pltpu.VMEM((GROUPS, L), jnp.float32),
          pltpu.VMEM((GROUPS, L), jnp.float32),
          pltpu.SemaphoreType.DMA,
      ],
  )
  def k(ri_pa, ri_pe, ri_na, ri_ne, app, ent, out_h,
        pa_i, pe_i, na_i, ne_i, pa_r, pe_r, na_r, ne_r,
        sp_flat, sn_flat, s_pos, s_neg, sem):
    wid = lax.axis_index("s") * NC + lax.axis_index("c")

    row0 = wid * NCHUNK
    pltpu.sync_copy(ri_pa.at[pl.ds(row0, NCHUNK)], pa_i)
    pltpu.sync_copy(ri_pe.at[pl.ds(row0, NCHUNK)], pe_i)
    pltpu.sync_copy(ri_na.at[pl.ds(row0, NCHUNK)], na_i)
    pltpu.sync_copy(ri_ne.at[pl.ds(row0, NCHUNK)], ne_i)

    copies = []
    for c in range(NCHUNK):
      dst = pl.ds(c * CHUNK, CHUNK)
      copies.append(pltpu.async_copy(app.at[pa_i.at[c]], pa_r.at[dst], sem))
      copies.append(pltpu.async_copy(ent.at[pe_i.at[c]], pe_r.at[dst], sem))
      copies.append(pltpu.async_copy(app.at[na_i.at[c]], na_r.at[dst], sem))
      copies.append(pltpu.async_copy(ent.at[ne_i.at[c]], ne_r.at[dst], sem))
    for cp in copies:
      cp.wait()

    lane = lax.iota(jnp.int32, L)
    lo = pl.ds(0, L)
    hi = pl.ds(L, L)

    def stage(r, _):
      sp_flat[pl.ds(r * STRIDE, L)] = (
          pa_r[r, lo] * pe_r[r, lo] + pa_r[r, hi] * pe_r[r, hi])
      sn_flat[pl.ds(r * STRIDE, L)] = (
          na_r[r, lo] * ne_r[r, lo] + na_r[r, hi] * ne_r[r, hi])
      return 0

    lax.fori_loop(0, BPW, stage, 0)

    def accum(g, _):
      base = (g * L + lane) * STRIDE
      accp = jnp.zeros((L,), jnp.float32)
      accn = jnp.zeros((L,), jnp.float32)
      for j in range(L):
        accp += plsc.load_gather(sp_flat, [base + j])
        accn += plsc.load_gather(sn_flat, [base + j])
      s_pos[g, :] = accp
      s_neg[g, :] = accn
      return 0

    lax.fori_loop(0, GROUPS, accum, 0)

    pltpu.sync_copy(s_pos, out_h.at[0, wid])
    pltpu.sync_copy(s_neg, out_h.at[1, wid])

  return k(ri[0], ri[1], ri[2], ri[3], app_rm, ent_rm)


def _tc_reduce(scores):
  def body(x_ref, o_ref):
    x = x_ref[...]
    row = lax.broadcasted_iota(jnp.int32, x.shape, 0)
    s = jnp.where(row < 128, x, -x)
    ls = jnp.minimum(s, 0.0) - jnp.log1p(jnp.exp(-jnp.abs(s)))
    o_ref[0, 0] = -jnp.sum(ls)

  out = pl.pallas_call(
      body,
      out_shape=jax.ShapeDtypeStruct((1, 1), jnp.float32),
      out_specs=pl.BlockSpec(memory_space=pltpu.SMEM),
  )(scores)
  return out[0, 0]


def kernel(pos_app, pos_entity, neg_app, neg_entity, app_emb, entity_emb):
  idx = [x.astype(jnp.int32).reshape(B // CHUNK, CHUNK)
         for x in (pos_app, pos_entity, neg_app, neg_entity)]
  scores = _sc_scores(idx, app_emb, entity_emb)
  return _tc_reduce(scores.reshape(2 * B // 128, 128))
